# baseline (speedup 1.0000x reference)
import jax
import jax.numpy as jnp
import numpy as np
from jax import lax
from jax.experimental import pallas as pl
from jax.experimental.pallas import tpu as pltpu

B = 8; H = 224; W = 224; PS = 14; PH = 16; PW = 16; P = 256; D = 1024; L = 4
NC = 12
KPAD = 640

_PREC = lax.Precision.HIGHEST


def _pool_matrix_1d(r: int) -> np.ndarray:
    idx = np.arange(PH)
    m = (np.abs(idx[:, None] - idx[None, :]) <= r // 2).astype(np.float32)
    return m / m.sum(axis=1, keepdims=True)


def _upsample_matrix(out_n: int, in_n: int) -> np.ndarray:
    xs = np.linspace(0.0, in_n - 1.0, out_n)
    x0 = np.clip(np.floor(xs).astype(np.int64), 0, in_n - 1)
    x1 = np.clip(x0 + 1, 0, in_n - 1)
    w = (xs - x0).astype(np.float32)
    a = np.zeros((out_n, in_n), np.float32)
    np.add.at(a, (np.arange(out_n), x0), 1.0 - w)
    np.add.at(a, (np.arange(out_n), x1), w)
    return a


_K3 = np.kron(_pool_matrix_1d(3), _pool_matrix_1d(3))
_K5 = np.kron(_pool_matrix_1d(5), _pool_matrix_1d(5))
_K35 = np.stack([_K3, _K5])
_AY = _upsample_matrix(H, PH)
_AX = _upsample_matrix(W, PW)


def _feat_kernel(p_ref, w_ref, k_ref, f_ref, sq_ref):
    x = p_ref[0]
    w = w_ref[0]
    z = jnp.dot(x, w, preferred_element_type=jnp.float32)
    mu = jnp.mean(z, axis=1, keepdims=True)
    var = jnp.mean((z - mu) ** 2, axis=1, keepdims=True)
    f = (z - mu) / jnp.sqrt(var + 1e-6)
    fb = f.astype(jnp.bfloat16)
    f_ref[0, 0, 0] = fb
    sq_ref[0, 0, 0] = jnp.sum(f * f, axis=1, keepdims=True)
    for i in range(2):
        pool = jnp.dot(k_ref[i], fb, preferred_element_type=jnp.float32)
        f_ref[i + 1, 0, 0] = pool.astype(jnp.bfloat16)
        sq_ref[i + 1, 0, 0] = jnp.sum(pool * pool, axis=1, keepdims=True)


def _mind2_kernel(fr_ref, sqr_ref, out_ref):
    fr = fr_ref[0].reshape(B * P, D)
    sqr = sqr_ref[0]
    for bq in range(B):
        fq = fr[bq * P:(bq + 1) * P]
        gt = lax.dot_general(fr, fq, (((1,), (1,)), ((), ())),
                             preferred_element_type=jnp.float32)
        d2 = sqr - 2.0 * gt.reshape(B, P, P)
        out_ref[0, bq] = jnp.min(d2, axis=1)


def _select_kernel(m2_ref, sq_ref, scores_ref, simg_ref):
    d2 = m2_ref[...] + sq_ref[...][:, :, None, :]
    d = jnp.sqrt(jnp.maximum(d2, 1e-12))
    bq = lax.broadcasted_iota(jnp.int32, d.shape, 1)
    br = lax.broadcasted_iota(jnp.int32, d.shape, 2)
    d = d + jnp.where(bq == br, jnp.float32(1e9), jnp.float32(0.0))
    min1 = jnp.full((NC, B, P), jnp.inf, jnp.float32)
    min2 = jnp.full((NC, B, P), jnp.inf, jnp.float32)
    for j in range(B):
        v = d[:, :, j, :]
        new1 = jnp.minimum(min1, v)
        min2 = jnp.minimum(min2, jnp.maximum(min1, v))
        min1 = new1
    scores = jnp.mean((min1 + min2) * 0.5, axis=0)
    scores_ref[...] = scores
    simg_ref[...] = jnp.max(scores, axis=1, keepdims=True)


def _upsample_kernel(s_ref, ay_ref, ax_ref, out_ref):
    ay = ay_ref[...]
    ax = ax_ref[...]
    for b in range(B):
        t = jnp.dot(ay, s_ref[b], preferred_element_type=jnp.float32,
                    precision=_PREC)
        out_ref[b] = lax.dot_general(t, ax, (((1,), (1,)), ((), ())),
                                     preferred_element_type=jnp.float32,
                                     precision=_PREC)


def kernel(pixel_values, W_patch):
    patches = pixel_values.reshape(B, 3, PH, PS, PW, PS)
    patches = patches.transpose(0, 2, 4, 1, 3, 5).reshape(B, P, 3 * PS * PS)
    patches = jnp.pad(patches, ((0, 0), (0, 0), (0, KPAD - 3 * PS * PS)))
    patches = patches.astype(jnp.bfloat16)
    w_pad = jnp.pad(W_patch, ((0, 0), (0, KPAD - 3 * PS * PS), (0, 0)))
    w_pad = w_pad.astype(jnp.bfloat16)
    k35 = jnp.asarray(_K35, dtype=jnp.bfloat16)

    fb3, sq3 = pl.pallas_call(
        _feat_kernel,
        grid=(L, B),
        in_specs=[
            pl.BlockSpec((1, P, KPAD), lambda l, b: (b, 0, 0)),
            pl.BlockSpec((1, KPAD, D), lambda l, b: (l, 0, 0)),
            pl.BlockSpec((2, P, P), lambda l, b: (0, 0, 0)),
        ],
        out_specs=(
            pl.BlockSpec((3, 1, 1, P, D), lambda l, b: (0, l, b, 0, 0)),
            pl.BlockSpec((3, 1, 1, P, 1), lambda l, b: (0, l, b, 0, 0)),
        ),
        out_shape=(jax.ShapeDtypeStruct((3, L, B, P, D), jnp.bfloat16),
                   jax.ShapeDtypeStruct((3, L, B, P, 1), jnp.float32)),
        compiler_params=pltpu.CompilerParams(
            dimension_semantics=("parallel", "parallel")),
    )(patches, w_pad, k35)

    f12 = fb3.reshape(NC, B, P, D)
    sq12 = sq3.reshape(NC, B, P, 1)

    m2 = pl.pallas_call(
        _mind2_kernel,
        grid=(NC,),
        in_specs=[
            pl.BlockSpec((1, B, P, D), lambda c: (c, 0, 0, 0)),
            pl.BlockSpec((1, B, P, 1), lambda c: (c, 0, 0, 0)),
        ],
        out_specs=pl.BlockSpec((1, B, B, P), lambda c: (c, 0, 0, 0)),
        out_shape=jax.ShapeDtypeStruct((NC, B, B, P), jnp.float32),
        compiler_params=pltpu.CompilerParams(
            dimension_semantics=("parallel",)),
    )(f12, sq12)

    scores, simg = pl.pallas_call(
        _select_kernel,
        out_shape=(jax.ShapeDtypeStruct((B, P), jnp.float32),
                   jax.ShapeDtypeStruct((B, 1), jnp.float32)),
    )(m2, sq12.reshape(NC, B, P))

    spix = pl.pallas_call(
        _upsample_kernel,
        out_shape=jax.ShapeDtypeStruct((B, H, W), jnp.float32),
    )(scores.reshape(B, PH, PW), jnp.asarray(_AY), jnp.asarray(_AX))

    return simg.reshape(B), spix


# speedup vs baseline: 1.0800x; 1.0800x over previous
import jax
import jax.numpy as jnp
import numpy as np
from jax import lax
from jax.experimental import pallas as pl
from jax.experimental.pallas import tpu as pltpu

B = 8; H = 224; W = 224; PS = 14; PH = 16; PW = 16; P = 256; D = 1024; L = 4
NC = 12
KPAD = 640

_PREC = lax.Precision.HIGHEST


def _pool_matrix_1d(r: int) -> np.ndarray:
    idx = np.arange(PH)
    m = (np.abs(idx[:, None] - idx[None, :]) <= r // 2).astype(np.float32)
    return m / m.sum(axis=1, keepdims=True)


def _upsample_matrix(out_n: int, in_n: int) -> np.ndarray:
    xs = np.linspace(0.0, in_n - 1.0, out_n)
    x0 = np.clip(np.floor(xs).astype(np.int64), 0, in_n - 1)
    x1 = np.clip(x0 + 1, 0, in_n - 1)
    w = (xs - x0).astype(np.float32)
    a = np.zeros((out_n, in_n), np.float32)
    np.add.at(a, (np.arange(out_n), x0), 1.0 - w)
    np.add.at(a, (np.arange(out_n), x1), w)
    return a


_K3 = np.kron(_pool_matrix_1d(3), _pool_matrix_1d(3))
_K5 = np.kron(_pool_matrix_1d(5), _pool_matrix_1d(5))
_K35 = np.stack([_K3, _K5])
_AY = _upsample_matrix(H, PH)
_AX = _upsample_matrix(W, PW)


def _fused_kernel(p_ref, w_ref, k_ref, m2_ref, sq_ref, f1_scr):
    s = pl.program_id(0)

    @pl.when(s < L)
    def _feature_step():
        x = p_ref[...].reshape(B * P, KPAD)
        z = jnp.dot(x, w_ref[0], preferred_element_type=jnp.float32)
        mu = jnp.mean(z, axis=1, keepdims=True)
        var = jnp.mean((z - mu) ** 2, axis=1, keepdims=True)
        f = (z - mu) / jnp.sqrt(var + 1e-6)
        f1_scr[s] = f.astype(jnp.bfloat16).reshape(B, P, D)

    @pl.when(s >= L)
    def _combo_step():
        c = s - L
        f1 = f1_scr[lax.rem(c, L)].reshape(B * P, D)
        ksel = k_ref[lax.div(c, L)]
        pools, sqs = [], []
        for b in range(B):
            pb = jnp.dot(ksel, f1[b * P:(b + 1) * P],
                         preferred_element_type=jnp.float32)
            sqs.append(jnp.sum(pb * pb, axis=1, keepdims=True))
            pools.append(pb.astype(jnp.bfloat16))
        fc = jnp.concatenate(pools, axis=0)
        sq = jnp.stack(sqs, axis=0)
        sq_ref[0] = sq
        for bq in range(B):
            fq = fc[bq * P:(bq + 1) * P]
            gt = lax.dot_general(fc, fq, (((1,), (1,)), ((), ())),
                                 preferred_element_type=jnp.float32)
            d2 = sq - 2.0 * gt.reshape(B, P, P)
            m2_ref[0, bq] = jnp.min(d2, axis=1)


def _select_kernel(m2_ref, sq_ref, scores_ref, simg_ref):
    d2 = m2_ref[...] + sq_ref[...][:, :, None, :]
    d = jnp.sqrt(jnp.maximum(d2, 1e-12))
    bq = lax.broadcasted_iota(jnp.int32, d.shape, 1)
    br = lax.broadcasted_iota(jnp.int32, d.shape, 2)
    d = d + jnp.where(bq == br, jnp.float32(1e9), jnp.float32(0.0))
    min1 = jnp.full((NC, B, P), jnp.inf, jnp.float32)
    min2 = jnp.full((NC, B, P), jnp.inf, jnp.float32)
    for j in range(B):
        v = d[:, :, j, :]
        new1 = jnp.minimum(min1, v)
        min2 = jnp.minimum(min2, jnp.maximum(min1, v))
        min1 = new1
    scores = jnp.mean((min1 + min2) * 0.5, axis=0)
    scores_ref[...] = scores
    simg_ref[...] = jnp.max(scores, axis=1, keepdims=True)


def _upsample_kernel(s_ref, ay_ref, ax_ref, out_ref):
    ay = ay_ref[...]
    ax = ax_ref[...]
    for b in range(B):
        t = jnp.dot(ay, s_ref[b], preferred_element_type=jnp.float32,
                    precision=_PREC)
        out_ref[b] = lax.dot_general(t, ax, (((1,), (1,)), ((), ())),
                                     preferred_element_type=jnp.float32,
                                     precision=_PREC)


def kernel(pixel_values, W_patch):
    patches = pixel_values.reshape(B, 3, PH, PS, PW, PS)
    patches = patches.transpose(0, 2, 4, 1, 3, 5).reshape(B, P, 3 * PS * PS)
    patches = jnp.pad(patches, ((0, 0), (0, 0), (0, KPAD - 3 * PS * PS)))
    patches = patches.astype(jnp.bfloat16)
    w_pad = jnp.pad(W_patch, ((0, 0), (0, KPAD - 3 * PS * PS), (0, 0)))
    w_pad = w_pad.astype(jnp.bfloat16)
    kI35 = jnp.asarray(np.stack([np.eye(P, dtype=np.float32), _K3, _K5]),
                       dtype=jnp.bfloat16)

    m2, sq12 = pl.pallas_call(
        _fused_kernel,
        grid=(L + NC,),
        in_specs=[
            pl.BlockSpec((B, P, KPAD), lambda s: (0, 0, 0)),
            pl.BlockSpec((1, KPAD, D), lambda s: (jnp.minimum(s, L - 1), 0, 0)),
            pl.BlockSpec((3, P, P), lambda s: (0, 0, 0)),
        ],
        out_specs=(
            pl.BlockSpec((1, B, B, P), lambda s: (jnp.maximum(s - L, 0), 0, 0, 0)),
            pl.BlockSpec((1, B, P, 1), lambda s: (jnp.maximum(s - L, 0), 0, 0, 0)),
        ),
        out_shape=(jax.ShapeDtypeStruct((NC, B, B, P), jnp.float32),
                   jax.ShapeDtypeStruct((NC, B, P, 1), jnp.float32)),
        scratch_shapes=[pltpu.VMEM((L, B, P, D), jnp.bfloat16)],
        compiler_params=pltpu.CompilerParams(
            dimension_semantics=("arbitrary",)),
    )(patches, w_pad, kI35)

    scores, simg = pl.pallas_call(
        _select_kernel,
        out_shape=(jax.ShapeDtypeStruct((B, P), jnp.float32),
                   jax.ShapeDtypeStruct((B, 1), jnp.float32)),
    )(m2, sq12.reshape(NC, B, P))

    spix = pl.pallas_call(
        _upsample_kernel,
        out_shape=jax.ShapeDtypeStruct((B, H, W), jnp.float32),
    )(scores.reshape(B, PH, PW), jnp.asarray(_AY), jnp.asarray(_AX))

    return simg.reshape(B), spix


# speedup vs baseline: 1.0810x; 1.0009x over previous
import jax
import jax.numpy as jnp
import numpy as np
from jax import lax
from jax.experimental import pallas as pl
from jax.experimental.pallas import tpu as pltpu

B = 8; H = 224; W = 224; PS = 14; PH = 16; PW = 16; P = 256; D = 1024; L = 4
NC = 12
KPAD = 640

_PREC = lax.Precision.HIGHEST


def _pool_matrix_1d(r: int) -> np.ndarray:
    idx = np.arange(PH)
    m = (np.abs(idx[:, None] - idx[None, :]) <= r // 2).astype(np.float32)
    return m / m.sum(axis=1, keepdims=True)


def _upsample_matrix(out_n: int, in_n: int) -> np.ndarray:
    xs = np.linspace(0.0, in_n - 1.0, out_n)
    x0 = np.clip(np.floor(xs).astype(np.int64), 0, in_n - 1)
    x1 = np.clip(x0 + 1, 0, in_n - 1)
    w = (xs - x0).astype(np.float32)
    a = np.zeros((out_n, in_n), np.float32)
    np.add.at(a, (np.arange(out_n), x0), 1.0 - w)
    np.add.at(a, (np.arange(out_n), x1), w)
    return a


_K3 = np.kron(_pool_matrix_1d(3), _pool_matrix_1d(3))
_K5 = np.kron(_pool_matrix_1d(5), _pool_matrix_1d(5))
_K35 = np.stack([_K3, _K5])
_AY = _upsample_matrix(H, PH)
_AX = _upsample_matrix(W, PW)


def _fused_kernel(p_ref, w_ref, k_ref, m2_ref, sq_ref, f1_scr):
    s = pl.program_id(0)

    @pl.when(s < L)
    def _feature_step():
        x = p_ref[...].reshape(B * P, KPAD)
        z = jnp.dot(x, w_ref[0], preferred_element_type=jnp.float32)
        mu = jnp.mean(z, axis=1, keepdims=True)
        var = jnp.mean((z - mu) ** 2, axis=1, keepdims=True)
        f = (z - mu) / jnp.sqrt(var + 1e-6)
        f1_scr[s] = f.astype(jnp.bfloat16).reshape(B, P, D)

    @pl.when(s >= L)
    def _combo_step():
        c = s - L
        f1 = f1_scr[lax.rem(c, L)].reshape(B * P, D)
        ksel = k_ref[lax.div(c, L)]
        pools, sqs = [], []
        for b in range(B):
            pb = jnp.dot(ksel, f1[b * P:(b + 1) * P],
                         preferred_element_type=jnp.float32)
            sqs.append(jnp.sum(pb * pb, axis=1, keepdims=True))
            pools.append(pb.astype(jnp.bfloat16))
        fc = jnp.concatenate(pools, axis=0)
        sq = jnp.stack(sqs, axis=0)
        sq_ref[0] = sq
        for bq in range(B):
            fq = fc[bq * P:(bq + 1) * P]
            gt = lax.dot_general(fc, fq, (((1,), (1,)), ((), ())),
                                 preferred_element_type=jnp.float32)
            d2 = sq - 2.0 * gt.reshape(B, P, P)
            m2_ref[0, bq] = jnp.min(d2, axis=1)


def _select_kernel(m2_ref, sq_ref, scores_ref, simg_ref):
    d2 = m2_ref[...] + sq_ref[...][:, :, None, :]
    d = jnp.sqrt(jnp.maximum(d2, 1e-12))
    bq = lax.broadcasted_iota(jnp.int32, d.shape, 1)
    br = lax.broadcasted_iota(jnp.int32, d.shape, 2)
    d = d + jnp.where(bq == br, jnp.float32(1e9), jnp.float32(0.0))
    min1 = jnp.full((NC, B, P), jnp.inf, jnp.float32)
    min2 = jnp.full((NC, B, P), jnp.inf, jnp.float32)
    for j in range(B):
        v = d[:, :, j, :]
        new1 = jnp.minimum(min1, v)
        min2 = jnp.minimum(min2, jnp.maximum(min1, v))
        min1 = new1
    scores = jnp.mean((min1 + min2) * 0.5, axis=0)
    scores_ref[...] = scores
    simg_ref[...] = jnp.max(scores, axis=1, keepdims=True)


def _upsample_kernel(s_ref, ay_ref, ax_ref, out_ref):
    ay = ay_ref[...]
    ax = ax_ref[...]
    for b in range(B):
        t = jnp.dot(ay, s_ref[b], preferred_element_type=jnp.float32,
                    precision=_PREC)
        out_ref[b] = lax.dot_general(t, ax, (((1,), (1,)), ((), ())),
                                     preferred_element_type=jnp.float32,
                                     precision=_PREC)


def kernel(pixel_values, W_patch):
    patches = pixel_values.astype(jnp.bfloat16).reshape(B, 3, PH, PS, PW, PS)
    patches = patches.transpose(0, 2, 4, 1, 3, 5).reshape(B, P, 3 * PS * PS)
    patches = jnp.pad(patches, ((0, 0), (0, 0), (0, KPAD - 3 * PS * PS)))
    w_pad = jnp.pad(W_patch, ((0, 0), (0, KPAD - 3 * PS * PS), (0, 0)))
    w_pad = w_pad.astype(jnp.bfloat16)
    kI35 = jnp.asarray(np.stack([np.eye(P, dtype=np.float32), _K3, _K5]),
                       dtype=jnp.bfloat16)

    m2, sq12 = pl.pallas_call(
        _fused_kernel,
        grid=(L + NC,),
        in_specs=[
            pl.BlockSpec((B, P, KPAD), lambda s: (0, 0, 0)),
            pl.BlockSpec((1, KPAD, D), lambda s: (jnp.minimum(s, L - 1), 0, 0)),
            pl.BlockSpec((3, P, P), lambda s: (0, 0, 0)),
        ],
        out_specs=(
            pl.BlockSpec((1, B, B, P), lambda s: (jnp.maximum(s - L, 0), 0, 0, 0)),
            pl.BlockSpec((1, B, P, 1), lambda s: (jnp.maximum(s - L, 0), 0, 0, 0)),
        ),
        out_shape=(jax.ShapeDtypeStruct((NC, B, B, P), jnp.float32),
                   jax.ShapeDtypeStruct((NC, B, P, 1), jnp.float32)),
        scratch_shapes=[pltpu.VMEM((L, B, P, D), jnp.bfloat16)],
        compiler_params=pltpu.CompilerParams(
            dimension_semantics=("arbitrary",)),
    )(patches, w_pad, kI35)

    scores, simg = pl.pallas_call(
        _select_kernel,
        out_shape=(jax.ShapeDtypeStruct((B, P), jnp.float32),
                   jax.ShapeDtypeStruct((B, 1), jnp.float32)),
    )(m2, sq12.reshape(NC, B, P))

    spix = pl.pallas_call(
        _upsample_kernel,
        out_shape=jax.ShapeDtypeStruct((B, H, W), jnp.float32),
    )(scores.reshape(B, PH, PW), jnp.asarray(_AY), jnp.asarray(_AX))

    return simg.reshape(B), spix
